# Initial kernel scaffold; baseline (speedup 1.0000x reference)
import jax
import jax.numpy as jnp
from jax.experimental import pallas as pl


def kernel(token_ids, table):
    raise NotImplementedError("write your pallas kernel here")


# speedup vs baseline: 1.4525x; 1.4525x over previous
import functools

import jax
import jax.numpy as jnp
from jax import lax
from jax.experimental import pallas as pl
from jax.experimental.pallas import tpu as pltpu
from jax.experimental.pallas import tpu_sc as plsc

VOCAB = 256
DIM = 768
NC = 2
NS = 16
NW = NC * NS
CHUNK = 64
NBUF = 2


@functools.lru_cache(maxsize=None)
def _make_emb(B: int):
    BPW = B // NW
    NCHUNK = BPW // CHUNK
    NITER = NCHUNK // NBUF
    mesh = plsc.VectorSubcoreMesh(core_axis_name="c", subcore_axis_name="s")

    @functools.partial(
        pl.kernel,
        mesh=mesh,
        out_type=jax.ShapeDtypeStruct((B, DIM), jnp.float32),
        scratch_types=[
            pltpu.VMEM((NCHUNK, CHUNK), jnp.int32),
            pltpu.VMEM((NBUF, CHUNK, DIM), jnp.float32),
            pltpu.SemaphoreType.DMA,
            pltpu.SemaphoreType.DMA,
            pltpu.SemaphoreType.DMA,
            pltpu.SemaphoreType.DMA,
        ],
    )
    def emb(ids_hbm, table_hbm, out_hbm, idx_v, rows_v, g0, g1, s0, s1):
        gsem = (g0, g1)
        ssem = (s0, s1)
        wid = lax.axis_index("s") * NC + lax.axis_index("c")
        base = wid * BPW
        pltpu.sync_copy(ids_hbm.at[wid], idx_v)

        def gather(c, b):
            return pltpu.make_async_copy(
                table_hbm.at[idx_v.at[c]], rows_v.at[b], gsem[b])

        def scatter(c, b):
            return pltpu.make_async_copy(
                rows_v.at[b], out_hbm.at[pl.ds(base + c * CHUNK, CHUNK)],
                ssem[b])

        for b in range(NBUF):
            gather(b, b).start()

        def body(i, carry):
            for b in range(NBUF):
                c = i * NBUF + b
                gather(c, b).wait()
                scatter(c, b).start()
                nxt = c + NBUF

                @pl.when(nxt < NCHUNK)
                def _prefetch():
                    scatter(c, b).wait()
                    gather(nxt, b).start()
            return carry

        lax.fori_loop(0, NITER, body, 0)
        for b in range(NBUF):
            scatter(0, b).wait()

    return emb


def kernel(token_ids, table):
    batch, seq = token_ids.shape
    B = batch * seq
    ids = token_ids.astype(jnp.int32).reshape(NW, B // NW // CHUNK, CHUNK)
    out = _make_emb(B)(ids, table)
    return out.reshape(batch, seq, DIM)


# speedup vs baseline: 1.4546x; 1.0015x over previous
import functools

import jax
import jax.numpy as jnp
from jax import lax
from jax.experimental import pallas as pl
from jax.experimental.pallas import tpu as pltpu
from jax.experimental.pallas import tpu_sc as plsc

VOCAB = 256
DIM = 768
NC = 2
NS = 16
NW = NC * NS
CHUNK = 32
NBUF = 4


@functools.lru_cache(maxsize=None)
def _make_emb(B: int):
    BPW = B // NW
    NCHUNK = BPW // CHUNK
    NITER = NCHUNK // NBUF
    mesh = plsc.VectorSubcoreMesh(core_axis_name="c", subcore_axis_name="s")

    @functools.partial(
        pl.kernel,
        mesh=mesh,
        out_type=jax.ShapeDtypeStruct((B, DIM), jnp.float32),
        scratch_types=[
            pltpu.VMEM((BPW,), jnp.int32),
            pltpu.VMEM((NBUF, CHUNK, DIM), jnp.float32),
        ] + [pltpu.SemaphoreType.DMA] * (2 * NBUF),
    )
    def emb(ids_hbm, table_hbm, out_hbm, idx_v, rows_v, *sems):
        gsem = sems[:NBUF]
        ssem = sems[NBUF:]
        wid = lax.axis_index("s") * NC + lax.axis_index("c")
        base = wid * BPW
        pltpu.sync_copy(ids_hbm.at[wid], idx_v)

        def gather(c, b):
            return pltpu.make_async_copy(
                table_hbm.at[idx_v.at[pl.ds(c * CHUNK, CHUNK)]],
                rows_v.at[b], gsem[b])

        def scatter(c, b):
            return pltpu.make_async_copy(
                rows_v.at[b], out_hbm.at[pl.ds(base + c * CHUNK, CHUNK)],
                ssem[b])

        for b in range(NBUF):
            gather(b, b).start()

        def body(i, carry):
            for b in range(NBUF):
                c = i * NBUF + b
                gather(c, b).wait()
                scatter(c, b).start()
                nxt = c + NBUF

                @pl.when(nxt < NCHUNK)
                def _prefetch():
                    scatter(c, b).wait()
                    gather(nxt, b).start()
            return carry

        lax.fori_loop(0, NITER, body, 0)
        for b in range(NBUF):
            scatter(0, b).wait()

    return emb


def kernel(token_ids, table):
    batch, seq = token_ids.shape
    B = batch * seq
    ids = token_ids.astype(jnp.int32).reshape(NW, B // NW)
    out = _make_emb(B)(ids, table)
    return out.reshape(batch, seq, DIM)


# speedup vs baseline: 1.9150x; 1.3165x over previous
import functools

import jax
import jax.numpy as jnp
from jax import lax
from jax.experimental import pallas as pl
from jax.experimental.pallas import tpu as pltpu
from jax.experimental.pallas import tpu_sc as plsc

VOCAB = 256
DIM = 768
NC = 2
NS = 16
NW = NC * NS
NIB = 4
NRB = 2


@functools.lru_cache(maxsize=None)
def _make_emb(batch: int, seq: int):
    IPW = batch // NW
    mesh = plsc.VectorSubcoreMesh(core_axis_name="c", subcore_axis_name="s")

    @functools.partial(
        pl.kernel,
        mesh=mesh,
        out_type=jax.ShapeDtypeStruct((batch, seq, DIM), jnp.float32),
        scratch_types=[
            pltpu.VMEM((NIB, seq), jnp.int32),
            pltpu.VMEM((NRB, seq, DIM), jnp.float32),
        ] + [pltpu.SemaphoreType.DMA] * (NIB + 2 * NRB),
        compiler_params=pltpu.CompilerParams(use_tc_tiling_on_sc=True),
    )
    def emb(ids_hbm, table_hbm, out_hbm, idx_v, rows_v, *sems):
        isem = sems[:NIB]
        gsem = sems[NIB:NIB + NRB]
        wsem = sems[NIB + NRB:]
        wid = lax.axis_index("s") * NC + lax.axis_index("c")
        base = wid * IPW

        def idx_load(j, ib):
            return pltpu.make_async_copy(
                ids_hbm.at[base + j], idx_v.at[ib], isem[ib])

        def gather(ib, rb):
            return pltpu.make_async_copy(
                table_hbm.at[idx_v.at[ib]], rows_v.at[rb], gsem[rb])

        def write(j, rb):
            return pltpu.make_async_copy(
                rows_v.at[rb], out_hbm.at[base + j], wsem[rb])

        for k in range(NIB):
            idx_load(k, k).start()
        for k in range(NRB):
            idx_load(k, k).wait()
            gather(k, k).start()

        def body(i, carry):
            for k in range(NIB):
                j = i * NIB + k
                rb = k % NRB
                gather(k, rb).wait()
                write(j, rb).start()

                @pl.when(j + NIB < IPW)
                def _prefetch_ids():
                    idx_load(j + NIB, k).start()

                @pl.when(j + NRB < IPW)
                def _next_gather():
                    write(j, rb).wait()
                    idx_load(0, (k + NRB) % NIB).wait()
                    gather((k + NRB) % NIB, rb).start()
            return carry

        lax.fori_loop(0, IPW // NIB, body, 0)
        for rb in range(NRB):
            write(0, rb).wait()

    return emb


def kernel(token_ids, table):
    batch, seq = token_ids.shape
    ids = token_ids.astype(jnp.int32)
    out = _make_emb(batch, seq)(ids, table)
    tail_start = (seq // 8) * 8
    if tail_start < seq:
        tail = jnp.take(table, ids[:, tail_start:], axis=0)
        out = out.at[:, tail_start:].set(tail)
    return out


# speedup vs baseline: 2.0053x; 1.0472x over previous
import functools

import jax
import jax.numpy as jnp
from jax import lax
from jax.experimental import pallas as pl
from jax.experimental.pallas import tpu as pltpu
from jax.experimental.pallas import tpu_sc as plsc

VOCAB = 256
DIM = 768
NC = 2
NS = 16
NW = NC * NS
NIB = 4
NRB = 2


@functools.lru_cache(maxsize=None)
def _make_emb(batch: int, seq: int):
    IPW = batch // NW
    mesh = plsc.VectorSubcoreMesh(core_axis_name="c", subcore_axis_name="s")

    @functools.partial(
        pl.kernel,
        mesh=mesh,
        out_type=jax.ShapeDtypeStruct((batch, seq, DIM), jnp.float32),
        scratch_types=[
            pltpu.VMEM((NIB, seq), jnp.int32),
            pltpu.VMEM((NRB, seq, DIM), jnp.float32),
        ] + [pltpu.SemaphoreType.DMA] * (NIB + 2 * NRB),
        compiler_params=pltpu.CompilerParams(use_tc_tiling_on_sc=True),
    )
    def emb(ids_hbm, table_hbm, out_hbm, idx_v, rows_v, *sems):
        isem = sems[:NIB]
        gsem = sems[NIB:NIB + NRB]
        wsem = sems[NIB + NRB:]
        wid = lax.axis_index("s") * NC + lax.axis_index("c")
        base = wid * IPW

        def idx_load(j, ib):
            return pltpu.make_async_copy(
                ids_hbm.at[base + j], idx_v.at[ib], isem[ib])

        def gather(ib, rb):
            return pltpu.make_async_copy(
                table_hbm.at[idx_v.at[ib]], rows_v.at[rb], gsem[rb])

        def write(j, rb):
            return pltpu.make_async_copy(
                rows_v.at[rb], out_hbm.at[base + j], wsem[rb])

        for k in range(NIB):
            idx_load(k, k).start()
        for k in range(NRB):
            idx_load(k, k).wait()
            gather(k, k).start()

        def body(i, carry):
            for k in range(NIB):
                j = i * NIB + k
                rb = k % NRB
                gather(k, rb).wait()
                write(j, rb).start()

                @pl.when(j + NIB < IPW)
                def _prefetch_ids():
                    idx_load(j + NIB, k).start()

                @pl.when(j + NRB < IPW)
                def _next_gather():
                    write(j, rb).wait()
                    idx_load(0, (k + NRB) % NIB).wait()
                    gather((k + NRB) % NIB, rb).start()
            return carry

        lax.fori_loop(0, IPW // NIB, body, 0)
        for rb in range(NRB):
            write(0, rb).wait()

    return emb


@functools.lru_cache(maxsize=None)
def _make_tail_fix(batch: int, seq: int):
    t0 = (seq // 8) * 8
    ntail = seq - t0
    BB = 512

    def body(ids_ref, table_ref, big_ref, out_ref):
        del big_ref
        ids = ids_ref[:, t0:seq]
        oh = (ids[..., None] == jax.lax.broadcasted_iota(
            jnp.int32, (1, 1, VOCAB), 2)).astype(jnp.float32)
        rows = jax.lax.dot_general(
            oh, table_ref[...], (((2,), (0,)), ((), ())),
            precision=jax.lax.Precision.HIGHEST)
        out_ref[:, :ntail, :] = rows
        out_ref[:, ntail:, :] = jnp.zeros((BB, 8 - ntail, DIM), jnp.float32)

    return pl.pallas_call(
        body,
        grid=(batch // BB,),
        in_specs=[
            pl.BlockSpec((BB, seq), lambda i: (i, 0)),
            pl.BlockSpec((VOCAB, DIM), lambda i: (0, 0)),
            pl.BlockSpec(memory_space=pltpu.MemorySpace.HBM),
        ],
        out_specs=pl.BlockSpec((BB, 8, DIM), lambda i: (i, seq // 8, 0)),
        out_shape=jax.ShapeDtypeStruct((batch, seq, DIM), jnp.float32),
        input_output_aliases={2: 0},
    )


def kernel(token_ids, table):
    batch, seq = token_ids.shape
    ids = token_ids.astype(jnp.int32)
    out = _make_emb(batch, seq)(ids, table)
    if seq % 8:
        out = _make_tail_fix(batch, seq)(ids, table, out)
    return out
